# baseline (speedup 1.0000x reference)
import functools

import jax
import jax.numpy as jnp
from jax import lax
from jax.experimental import pallas as pl
from jax.experimental.pallas import tpu as pltpu
from jax.experimental.pallas import tpu_sc as plsc

_NC = 2
_NS = 16
_NW = _NC * _NS
_CHUNK = 2048
_C = 24


def _sc_body(n_tokens, yp_hbm, idx_hbm, bt_hbm, out_hbm,
             yb0, yb1, ib0, ib1, btb, sb, pb,
             sy0, sy1, si0, si1):
    tpw = n_tokens // _NW
    nchunk = tpw // _CHUNK

    wid = lax.axis_index("s") * _NC + lax.axis_index("c")
    woff = wid * tpw

    ybufs = (yb0, yb1)
    ibufs = (ib0, ib1)
    ysems = (sy0, sy1)
    isems = (si0, si1)

    pltpu.sync_copy(bt_hbm, btb)

    zeros = jnp.zeros((16,), jnp.float32)

    def _zero(k, carry):
        sb[pl.ds(k * 16, 16)] = zeros
        return carry

    lax.fori_loop(0, 16 * 576 // 16, _zero, 0)

    laneoff = lax.iota(jnp.int32, 16) * 576
    i24 = lax.iota(jnp.int32, 16) * _C

    def _start(ch, b):
        pltpu.make_async_copy(
            yp_hbm.at[pl.ds((woff + ch * _CHUNK) * _C, _CHUNK * _C)],
            ybufs[b], ysems[b]).start()
        pltpu.make_async_copy(
            idx_hbm.at[pl.ds(woff + ch * _CHUNK, _CHUNK)],
            ibufs[b], isems[b]).start()

    def _wait(b):
        pltpu.make_async_copy(
            yp_hbm.at[pl.ds(0, _CHUNK * _C)], ybufs[b], ysems[b]).wait()
        pltpu.make_async_copy(
            idx_hbm.at[pl.ds(0, _CHUNK)], ibufs[b], isems[b]).wait()

    def _process(b):
        ybuf = ybufs[b]
        ibuf = ibufs[b]

        def _group(g, carry):
            idxv = ibuf[pl.ds(g * 16, 16)]
            dbase = idxv * _C + laneoff
            gb = g * (16 * _C)
            for j in range(_C):
                y = plsc.load_gather(ybuf, [i24 + (gb + j)])
                plsc.addupdate_scatter(sb, [dbase + j], y)
            return carry

        lax.fori_loop(0, _CHUNK // 16, _group, 0)

    _start(0, 0)

    def _outer(h, carry):
        for b in range(2):
            ch = h * 2 + b
            nxt = ch + 1

            @pl.when(nxt < nchunk)
            def _():
                _start(nxt, (b + 1) % 2)

            _wait(b)
            _process(b)
        return carry

    lax.fori_loop(0, nchunk // 2, _outer, 0)

    def _red_outer(l, acc):
        def _red_inner(k, acc2):
            return acc2 + (sb[pl.ds(l * 576 + k * 16, 16)] *
                           btb[pl.ds(k * 16, 16)])
        return lax.fori_loop(0, 576 // 16, _red_inner, acc)

    acc = lax.fori_loop(0, 16, _red_outer, jnp.zeros((16,), jnp.float32))
    pb[...] = acc
    pltpu.sync_copy(pb, out_hbm.at[wid])


def kernel(y_true, y_pred, B):
    n = y_true.shape[0] * y_true.shape[1]
    idx = y_true.reshape(-1).astype(jnp.int32)
    yp = y_pred.reshape(-1)
    btflat = B.T.reshape(-1)

    mesh = plsc.VectorSubcoreMesh(core_axis_name="c", subcore_axis_name="s")
    run = pl.kernel(
        functools.partial(_sc_body, n),
        out_type=jax.ShapeDtypeStruct((_NW, 16), jnp.float32),
        mesh=mesh,
        scratch_types=[
            pltpu.VMEM((_CHUNK * _C,), jnp.float32),
            pltpu.VMEM((_CHUNK * _C,), jnp.float32),
            pltpu.VMEM((_CHUNK,), jnp.int32),
            pltpu.VMEM((_CHUNK,), jnp.int32),
            pltpu.VMEM((576,), jnp.float32),
            pltpu.VMEM((16 * 576,), jnp.float32),
            pltpu.VMEM((16,), jnp.float32),
            pltpu.SemaphoreType.DMA,
            pltpu.SemaphoreType.DMA,
            pltpu.SemaphoreType.DMA,
            pltpu.SemaphoreType.DMA,
        ],
        compiler_params=pltpu.CompilerParams(needs_layout_passes=False),
    )
    out = run(yp, idx, btflat)
    return jnp.sum(out)


# speedup vs baseline: 1.2426x; 1.2426x over previous
import functools

import jax
import jax.numpy as jnp
from jax import lax
from jax.experimental import pallas as pl
from jax.experimental.pallas import tpu as pltpu
from jax.experimental.pallas import tpu_sc as plsc

_NC = 2
_NS = 16
_NW = _NC * _NS
_CHUNK = 2048
_C = 24
_NACC = 4


def _sc_body(n_tokens, yp_hbm, idx_hbm, bt_hbm, out_hbm,
             yb0, yb1, ib0, ib1, btb, pb,
             sy0, sy1, si0, si1):
    tpw = n_tokens // _NW
    nchunk = tpw // _CHUNK

    wid = lax.axis_index("s") * _NC + lax.axis_index("c")
    woff = wid * tpw

    ybufs = (yb0, yb1)
    ibufs = (ib0, ib1)
    ysems = (sy0, sy1)
    isems = (si0, si1)

    pltpu.sync_copy(bt_hbm, btb)

    i24 = lax.iota(jnp.int32, 16) * _C

    def _start(ch, b):
        pltpu.make_async_copy(
            yp_hbm.at[pl.ds((woff + ch * _CHUNK) * _C, _CHUNK * _C)],
            ybufs[b], ysems[b]).start()
        pltpu.make_async_copy(
            idx_hbm.at[pl.ds(woff + ch * _CHUNK, _CHUNK)],
            ibufs[b], isems[b]).start()

    def _wait(b):
        pltpu.make_async_copy(
            yp_hbm.at[pl.ds(0, _CHUNK * _C)], ybufs[b], ysems[b]).wait()
        pltpu.make_async_copy(
            idx_hbm.at[pl.ds(0, _CHUNK)], ibufs[b], isems[b]).wait()

    def _process(b, accs):
        ybuf = ybufs[b]
        ibuf = ibufs[b]

        def _group(g, accs):
            accs = list(accs)
            idxv = ibuf[pl.ds(g * 16, 16)]
            wbase = idxv * _C
            ybase = i24 + g * (16 * _C)
            for j in range(_C):
                w = plsc.load_gather(btb, [wbase + j])
                y = plsc.load_gather(ybuf, [ybase + j])
                accs[j % _NACC] = accs[j % _NACC] + w * y
            return tuple(accs)

        return lax.fori_loop(0, _CHUNK // 16, _group, accs)

    _start(0, 0)

    def _outer(h, accs):
        for b in range(2):
            ch = h * 2 + b
            nxt = ch + 1

            @pl.when(nxt < nchunk)
            def _():
                _start(nxt, (b + 1) % 2)

            _wait(b)
            accs = _process(b, accs)
        return accs

    zeros = jnp.zeros((16,), jnp.float32)
    accs = lax.fori_loop(0, nchunk // 2, _outer, (zeros,) * _NACC)

    total = accs[0]
    for a in accs[1:]:
        total = total + a
    pb[...] = total
    pltpu.sync_copy(pb, out_hbm.at[wid])


def kernel(y_true, y_pred, B):
    n = y_true.shape[0] * y_true.shape[1]
    idx = y_true.reshape(-1).astype(jnp.int32)
    yp = y_pred.reshape(-1)
    btflat = B.T.reshape(-1)

    mesh = plsc.VectorSubcoreMesh(core_axis_name="c", subcore_axis_name="s")
    run = pl.kernel(
        functools.partial(_sc_body, n),
        out_type=jax.ShapeDtypeStruct((_NW, 16), jnp.float32),
        mesh=mesh,
        scratch_types=[
            pltpu.VMEM((_CHUNK * _C,), jnp.float32),
            pltpu.VMEM((_CHUNK * _C,), jnp.float32),
            pltpu.VMEM((_CHUNK,), jnp.int32),
            pltpu.VMEM((_CHUNK,), jnp.int32),
            pltpu.VMEM((576,), jnp.float32),
            pltpu.VMEM((16,), jnp.float32),
            pltpu.SemaphoreType.DMA,
            pltpu.SemaphoreType.DMA,
            pltpu.SemaphoreType.DMA,
            pltpu.SemaphoreType.DMA,
        ],
        compiler_params=pltpu.CompilerParams(needs_layout_passes=False),
    )
    out = run(yp, idx, btflat)
    return jnp.sum(out)


# speedup vs baseline: 14.3018x; 11.5096x over previous
import jax
import jax.numpy as jnp
from jax.experimental import pallas as pl

_R = 16384
_TT = 200
_C = 24


def _score_kernel(idx_ref, yp_ref, b_ref, out_ref):
    step = pl.program_id(0)

    idx = idx_ref[...].reshape(1, _R)
    yp = yp_ref[...]
    b = b_ref[...]

    iota = jax.lax.broadcasted_iota(jnp.int32, (_C, _R), 0)
    onehot = (idx == iota).astype(jnp.float32)
    s = jax.lax.dot_general(yp, onehot, (((1,), (1,)), ((), ())),
                            preferred_element_type=jnp.float32)
    partial = jnp.sum(s * b)

    @pl.when(step == 0)
    def _():
        out_ref[...] = jnp.zeros_like(out_ref)

    out_ref[...] = out_ref[...] + partial


def kernel(y_true, y_pred, B):
    ypt = y_pred.transpose(1, 2, 0).reshape(_TT * _C, _R)
    idx = y_true.T.reshape(_TT, 1, _R).astype(jnp.int32)

    out = pl.pallas_call(
        _score_kernel,
        grid=(_TT,),
        in_specs=[
            pl.BlockSpec((1, 1, _R), lambda i: (i, 0, 0)),
            pl.BlockSpec((_C, _R), lambda i: (i, 0)),
            pl.BlockSpec((_C, _C), lambda i: (0, 0)),
        ],
        out_specs=pl.BlockSpec((1, 1), lambda i: (0, 0)),
        out_shape=jax.ShapeDtypeStruct((1, 1), jnp.float32),
    )(idx, ypt, B)
    return out[0, 0]


# speedup vs baseline: 25.1548x; 1.7589x over previous
import jax
import jax.numpy as jnp
from jax.experimental import pallas as pl

_R = 16384
_TT = 200
_C = 24
_TB = 4


def _score_kernel(idx_ref, yp_ref, b_ref, out_ref):
    step = pl.program_id(0)

    b = b_ref[...]
    iota = jax.lax.broadcasted_iota(jnp.int32, (_C, _R), 0)

    partial = jnp.zeros((), jnp.float32)
    for u in range(_TB):
        idx = idx_ref[u].reshape(1, _R)
        yp = yp_ref[u * _C:(u + 1) * _C, :]
        onehot = (idx == iota).astype(jnp.float32)
        s = jax.lax.dot_general(yp, onehot, (((1,), (1,)), ((), ())),
                                preferred_element_type=jnp.float32)
        partial = partial + jnp.sum(s * b)

    @pl.when(step == 0)
    def _():
        out_ref[...] = jnp.zeros_like(out_ref)

    out_ref[...] = out_ref[...] + partial


def kernel(y_true, y_pred, B):
    ypt = y_pred.transpose(1, 2, 0).reshape(_TT * _C, _R)
    idx = y_true.T.reshape(_TT, 1, _R).astype(jnp.int32)

    out = pl.pallas_call(
        _score_kernel,
        grid=(_TT // _TB,),
        in_specs=[
            pl.BlockSpec((_TB, 1, _R), lambda i: (i, 0, 0)),
            pl.BlockSpec((_TB * _C, _R), lambda i: (i, 0)),
            pl.BlockSpec((_C, _C), lambda i: (0, 0)),
        ],
        out_specs=pl.BlockSpec((1, 1), lambda i: (0, 0)),
        out_shape=jax.ShapeDtypeStruct((1, 1), jnp.float32),
    )(idx, ypt, B)
    return out[0, 0]


# speedup vs baseline: 28.0386x; 1.1146x over previous
import jax
import jax.numpy as jnp
from jax.experimental import pallas as pl

_R = 16384
_TT = 200
_C = 24
_TB = 8


def _score_kernel(idx_ref, yp_ref, b_ref, out_ref):
    step = pl.program_id(0)

    b = b_ref[...]
    iota = jax.lax.broadcasted_iota(jnp.int32, (_C, _R), 0)

    partial = jnp.zeros((), jnp.float32)
    for u in range(_TB):
        idx = idx_ref[u].reshape(1, _R)
        yp = yp_ref[u * _C:(u + 1) * _C, :]
        onehot = (idx == iota).astype(jnp.float32)
        s = jax.lax.dot_general(yp, onehot, (((1,), (1,)), ((), ())),
                                preferred_element_type=jnp.float32)
        partial = partial + jnp.sum(s * b)

    @pl.when(step == 0)
    def _():
        out_ref[...] = jnp.zeros_like(out_ref)

    out_ref[...] = out_ref[...] + partial


def kernel(y_true, y_pred, B):
    ypt = y_pred.transpose(1, 2, 0).reshape(_TT * _C, _R)
    idx = y_true.T.reshape(_TT, 1, _R).astype(jnp.int32)

    out = pl.pallas_call(
        _score_kernel,
        grid=(_TT // _TB,),
        in_specs=[
            pl.BlockSpec((_TB, 1, _R), lambda i: (i, 0, 0)),
            pl.BlockSpec((_TB * _C, _R), lambda i: (i, 0)),
            pl.BlockSpec((_C, _C), lambda i: (0, 0)),
        ],
        out_specs=pl.BlockSpec((1, 1), lambda i: (0, 0)),
        out_shape=jax.ShapeDtypeStruct((1, 1), jnp.float32),
    )(idx, ypt, B)
    return out[0, 0]
